# Initial kernel scaffold; baseline (speedup 1.0000x reference)
import jax
import jax.numpy as jnp
from jax.experimental import pallas as pl


def kernel(x, timestep, edge_index, l0_W, l0_b, tm_W, tm_b, d_gc_W, d_gc_b, d_tm_W, d_tm_b, d_lin_W, d_lin_b, d_ln_g, d_ln_b, u_gc_W, u_gc_b, u_tm_W, u_tm_b, u_lin_W, u_lin_b, u_ln_g, u_ln_b, out_W, out_b):
    raise NotImplementedError("write your pallas kernel here")


# speedup vs baseline: 5.2806x; 5.2806x over previous
import functools

import jax
import jax.numpy as jnp
from jax import lax
from jax.experimental import pallas as pl
from jax.experimental.pallas import tpu as pltpu
from jax.experimental.pallas import tpu_sc as plsc

N = 10000
E = 320000
F_IN = 128
H0 = 256
H1 = 512
T_DIM = 128

CW = 64
N_PAD = 10112
ROWS_N = N_PAD // 16
ROWS_E = 2560
E_PAD = ROWS_E * 128
EROWS_TILE = ROWS_E // 16
EROWS_WORKER = ROWS_E // 32

_f32 = jnp.float32


def _sc_mesh():
    return plsc.VectorSubcoreMesh(core_axis_name="c", subcore_axis_name="s",
                                  num_cores=2, num_subcores=16)


@functools.cache
def _get_deg_kernel():
    @functools.partial(
        pl.kernel,
        out_type=jax.ShapeDtypeStruct((2, N_PAD, 16), _f32),
        mesh=_sc_mesh(),
        scratch_types=[
            pltpu.VMEM((EROWS_WORKER, 128), jnp.int32),
            pltpu.VMEM((128, 16), _f32),
            pltpu.VMEM_SHARED((N_PAD, 16), _f32),
            pltpu.SemaphoreType.DMA,
        ],
        compiler_params=pltpu.CompilerParams(use_tc_tiling_on_sc=False),
    )
    def _deg_kernel(dst_hbm, ones_hbm, zeros_hbm, out_hbm, idx_v, ones_v,
                    acc, sem):
        cid = lax.axis_index("c")
        sid = lax.axis_index("s")
        wid = sid * 2 + cid
        row0 = sid * ROWS_N
        pltpu.sync_copy(zeros_hbm, acc.at[pl.ds(row0, ROWS_N)])
        pltpu.sync_copy(ones_hbm, ones_v)
        pltpu.sync_copy(dst_hbm.at[pl.ds(wid * EROWS_WORKER, EROWS_WORKER)],
                        idx_v)
        plsc.subcore_barrier()

        def body(j, carry):
            pltpu.sync_copy(ones_v, acc.at[idx_v.at[j]], add=True)
            return carry

        lax.fori_loop(0, EROWS_WORKER, body, 0)
        plsc.subcore_barrier()
        pltpu.sync_copy(acc.at[pl.ds(row0, ROWS_N)],
                        out_hbm.at[cid, pl.ds(row0, ROWS_N)])

    return _deg_kernel


@functools.cache
def _get_scatter(n_chunks):
    chunks_per_sc = n_chunks // 2

    @functools.partial(
        pl.kernel,
        out_type=[jax.ShapeDtypeStruct((N_PAD, CW), _f32)] * n_chunks,
        mesh=_sc_mesh(),
        scratch_types=[
            pltpu.VMEM((EROWS_TILE, 128), jnp.int32),
            pltpu.VMEM((EROWS_TILE, 128), jnp.int32),
            pltpu.VMEM((128, CW), _f32),
            pltpu.VMEM_SHARED((N_PAD, CW), _f32),
            pltpu.SemaphoreType.DMA,
        ],
        compiler_params=pltpu.CompilerParams(use_tc_tiling_on_sc=False),
    )
    def _scatter(src_hbm, dst_hbm, *rest):
        g_refs = rest[:n_chunks]
        zeros_hbm = rest[n_chunks]
        out_refs = rest[n_chunks + 1:n_chunks + 1 + n_chunks]
        src_v, dst_v, pay_v, acc, sem = rest[n_chunks + 1 + n_chunks:]

        cid = lax.axis_index("c")
        sid = lax.axis_index("s")
        row0 = sid * ROWS_N
        erow0 = sid * EROWS_TILE
        pltpu.sync_copy(src_hbm.at[pl.ds(erow0, EROWS_TILE)], src_v)
        pltpu.sync_copy(dst_hbm.at[pl.ds(erow0, EROWS_TILE)], dst_v)

        def run_chunk(g_hbm, out_hbm):
            pltpu.sync_copy(zeros_hbm, acc.at[pl.ds(row0, ROWS_N)])
            plsc.subcore_barrier()

            def body(j, carry):
                pltpu.async_copy(g_hbm.at[src_v.at[j]], pay_v, sem).wait()
                pltpu.sync_copy(pay_v, acc.at[dst_v.at[j]], add=True)
                return carry

            lax.fori_loop(0, EROWS_TILE, body, 0)
            plsc.subcore_barrier()
            pltpu.sync_copy(acc.at[pl.ds(row0, ROWS_N)],
                            out_hbm.at[pl.ds(row0, ROWS_N)])

        for k in range(chunks_per_sc):

            @pl.when(cid == 0)
            def _():
                run_chunk(g_refs[k], out_refs[k])

            @pl.when(cid == 1)
            def _():
                run_chunk(g_refs[chunks_per_sc + k],
                          out_refs[chunks_per_sc + k])

    return _scatter


def _mmT(a, w):
    return lax.dot_general(a, w, (((1,), (1,)), ((), ())),
                           preferred_element_type=_f32)


def _silu(v):
    return v * jax.nn.sigmoid(v)


def _minmax_body(x_ref, mn_ref, mx_ref):
    mn_ref[...] = jnp.min(x_ref[...]).reshape(1, 1)
    mx_ref[...] = jnp.max(x_ref[...]).reshape(1, 1)


BN = 1000
GRID = N // BN

NC_D = H1 // CW
NC_U = H0 // CW


def _pre_body(x_ref, ts_ref, mn_ref, mx_ref, dinv_ref, l0W, l0b, emb, tmW,
              tmb, dtmW, dtmb, utmW, utmb, dgcW, *outs):
    g_outs = outs[:NC_D]
    td, tu = outs[NC_D:]
    mn = mn_ref[...]
    mx = mx_ref[...]
    xs = (x_ref[...] - mn) * (1.0 / (mx - mn + 1e-12))
    h1 = _mmT(xs, l0W[...]) + l0b[...]
    e = ts_ref[...] * emb[...]
    sfull = jnp.concatenate([jnp.sin(e), jnp.cos(e)], axis=1)
    t = jax.nn.relu(_mmT(sfull, tmW[...]) + tmb[...])
    td[...] = _silu(_mmT(t, dtmW[...]) + dtmb[...])
    tu[...] = _silu(_mmT(t, utmW[...]) + utmb[...])
    gd = dinv_ref[...] * _mmT(h1, dgcW[...])
    for i in range(NC_D):
        g_outs[i][...] = gd[:, CW * i:CW * (i + 1)]


def _ln(v, g, b):
    m = jnp.mean(v, axis=1, keepdims=True)
    var = jnp.mean((v - m) ** 2, axis=1, keepdims=True)
    return (v - m) * lax.rsqrt(var + 1e-5) * g + b


def _mid_body(*refs):
    s = refs[:NC_D]
    g = refs[NC_D:2 * NC_D]
    dinv_ref, dgcb, td, dlinW, dlinb, dlng, dlnb, ugcW = \
        refs[2 * NC_D:2 * NC_D + 8]
    gu_outs = refs[2 * NC_D + 8:]
    dinv = dinv_ref[...]
    z = jnp.concatenate(
        [dinv * (s[i][...] + g[i][...]) for i in range(NC_D)],
        axis=1) + dgcb[...]
    z = _silu(z) + td[...]
    h2 = _ln(_silu(_mmT(z, dlinW[...]) + dlinb[...]), dlng[...], dlnb[...])
    gu = dinv * _mmT(h2, ugcW[...])
    for i in range(NC_U):
        gu_outs[i][...] = gu[:, CW * i:CW * (i + 1)]


def _post_body(*refs):
    s = refs[:NC_U]
    g = refs[NC_U:2 * NC_U]
    (dinv_ref, ugcb, tu, ulinW, ulinb, ulng, ulnb, outW, outb,
     out) = refs[2 * NC_U:]
    dinv = dinv_ref[...]
    z = jnp.concatenate(
        [dinv * (s[i][...] + g[i][...]) for i in range(NC_U)],
        axis=1) + ugcb[...]
    z = _silu(z) + tu[...]
    h3 = _ln(_silu(_mmT(z, ulinW[...]) + ulinb[...]), ulng[...], ulnb[...])
    logits = _mmT(h3, outW[...]) + outb[...]
    m = jnp.max(logits, axis=1, keepdims=True)
    ls = logits - m
    out[...] = ls - jnp.log(jnp.sum(jnp.exp(ls), axis=1, keepdims=True))


def _row_spec(d):
    return pl.BlockSpec((BN, d), lambda i: (i, 0))


def _full_spec(shape):
    nd = len(shape)
    return pl.BlockSpec(shape, lambda i: (0,) * nd)


def kernel(x, timestep, edge_index, l0_W, l0_b, tm_W, tm_b, d_gc_W, d_gc_b,
           d_tm_W, d_tm_b, d_lin_W, d_lin_b, d_ln_g, d_ln_b, u_gc_W, u_gc_b,
           u_tm_W, u_tm_b, u_lin_W, u_lin_b, u_ln_g, u_ln_b, out_W, out_b):
    src = edge_index[0]
    dst = edge_index[1]
    pad = E_PAD - E
    src2 = jnp.concatenate(
        [src, jnp.zeros((pad,), jnp.int32)]).reshape(ROWS_E, 128)
    dst2 = jnp.concatenate(
        [dst, jnp.full((pad,), N, jnp.int32)]).reshape(ROWS_E, 128)

    ones_pay = jnp.ones((128, 16), _f32)
    zeros_deg = jnp.zeros((ROWS_N, 16), _f32)
    zeros_s = jnp.zeros((ROWS_N, CW), _f32)

    degp = _get_deg_kernel()(dst2, ones_pay, zeros_deg)
    deg = degp[0, :N, 0] + degp[1, :N, 0] + 1.0
    dinv = lax.rsqrt(deg).reshape(N, 1)

    mn, mx = pl.pallas_call(
        _minmax_body,
        out_shape=[jax.ShapeDtypeStruct((1, 1), _f32)] * 2,
    )(x)

    ts2 = timestep.reshape(N, 1)
    emb = jnp.exp(jnp.arange(T_DIM // 2, dtype=_f32)
                  * (-jnp.log(10000.0) / (T_DIM // 2 - 1))).reshape(1, -1)

    b2 = lambda b: b.reshape(1, -1)

    pre_out = pl.pallas_call(
        _pre_body,
        grid=(GRID,),
        in_specs=[
            _row_spec(F_IN), _row_spec(1), _full_spec((1, 1)),
            _full_spec((1, 1)), _row_spec(1),
            _full_spec((H0, F_IN)), _full_spec((1, H0)),
            _full_spec((1, T_DIM // 2)),
            _full_spec((T_DIM, T_DIM)), _full_spec((1, T_DIM)),
            _full_spec((H1, T_DIM)), _full_spec((1, H1)),
            _full_spec((H0, T_DIM)), _full_spec((1, H0)),
            _full_spec((H1, H0)),
        ],
        out_specs=[_row_spec(CW)] * NC_D + [_row_spec(H1), _row_spec(H0)],
        out_shape=[jax.ShapeDtypeStruct((N, CW), _f32)] * NC_D
        + [jax.ShapeDtypeStruct((N, H1), _f32),
           jax.ShapeDtypeStruct((N, H0), _f32)],
    )(x, ts2, mn, mx, dinv, l0_W, b2(l0_b), emb, tm_W, b2(tm_b),
      d_tm_W, b2(d_tm_b), u_tm_W, b2(u_tm_b), d_gc_W)
    gd = pre_out[:NC_D]
    td, tu = pre_out[NC_D:]

    sd = _get_scatter(NC_D)(src2, dst2, *gd, zeros_s)

    gu = pl.pallas_call(
        _mid_body,
        grid=(GRID,),
        in_specs=[_row_spec(CW)] * (2 * NC_D) + [
            _row_spec(1), _full_spec((1, H1)), _row_spec(H1),
            _full_spec((H1, H1)), _full_spec((1, H1)),
            _full_spec((1, H1)), _full_spec((1, H1)),
            _full_spec((H0, H1)),
        ],
        out_specs=[_row_spec(CW)] * NC_U,
        out_shape=[jax.ShapeDtypeStruct((N, CW), _f32)] * NC_U,
    )(*[s[:N] for s in sd], *gd, dinv, b2(d_gc_b), td,
      d_lin_W, b2(d_lin_b), b2(d_ln_g), b2(d_ln_b), u_gc_W)

    su = _get_scatter(NC_U)(src2, dst2, *gu, zeros_s)

    out = pl.pallas_call(
        _post_body,
        grid=(GRID,),
        in_specs=[_row_spec(CW)] * (2 * NC_U) + [
            _row_spec(1), _full_spec((1, H0)), _row_spec(H0),
            _full_spec((H0, H0)), _full_spec((1, H0)),
            _full_spec((1, H0)), _full_spec((1, H0)),
            _full_spec((F_IN, H0)), _full_spec((1, F_IN)),
        ],
        out_specs=[_row_spec(F_IN)],
        out_shape=[jax.ShapeDtypeStruct((N, F_IN), _f32)],
    )(*[s[:N] for s in su], *gu, dinv, b2(u_gc_b), tu,
      u_lin_W, b2(u_lin_b), b2(u_ln_g), b2(u_ln_b), out_W, b2(out_b))

    return out[0]
